# baseline (speedup 1.0000x reference)
import functools

import jax
import jax.numpy as jnp
from jax import lax
from jax.experimental import pallas as pl
from jax.experimental.pallas import tpu as pltpu
from jax.experimental.pallas import tpu_sc as plsc

R, C = 16384, 200
D = 32
NW = 32
BI = 256
NBLK = R // BI
BPW = NBLK // NW
L = 16
NG = BI // L
GQ = 8
JB = 4
NQ = C // JB


def _sc_lookup(labels_t, tabx):
    mesh = plsc.VectorSubcoreMesh(core_axis_name="c", subcore_axis_name="s")

    @functools.partial(
        pl.kernel,
        mesh=mesh,
        out_type=jax.ShapeDtypeStruct((C, D, R), jnp.float32),
        scratch_types=[
            pltpu.VMEM((D * L,), jnp.float32),
            pltpu.VMEM((C, BI), jnp.int32),
            pltpu.VMEM((JB, D, BI), jnp.float32),
            pltpu.VMEM((JB, D, BI), jnp.float32),
            pltpu.SemaphoreType.DMA,
            pltpu.SemaphoreType.DMA,
        ],
    )
    def k(labels_hbm, tabx_hbm, out_hbm, tabx_v, slab_v, quad0, quad1,
          semo0, semo1):
        quad_b = (quad0, quad1)
        semo_b = (semo0, semo1)
        wid = lax.axis_index("s") * 2 + lax.axis_index("c")
        pltpu.sync_copy(tabx_hbm, tabx_v)
        dnums = lax.GatherDimensionNumbers(
            offset_dims=(), collapsed_slice_dims=(0,), start_index_map=(0,)
        )

        def pick(vd, lv):
            return lax.gather(
                vd, lv[:, None], dnums, slice_sizes=(1,),
                mode=lax.GatherScatterMode.PROMISE_IN_BOUNDS,
            )

        def jrow(quad_v, jj, j):
            def gquad(gq, carry):
                g0 = gq * GQ
                lvs = [
                    slab_v[j, pl.ds((g0 + u) * L, L)] for u in range(GQ)
                ]
                for d in range(D):
                    vd = tabx_v[pl.ds(d * L, L)]
                    for u in range(GQ):
                        quad_v[jj, d, pl.ds((g0 + u) * L, L)] = (
                            pick(vd, lvs[u])
                        )
                return carry

            lax.fori_loop(0, NG // GQ, gquad, 0)

        for blk in range(BPW):
            i0 = (wid * BPW + blk) * BI
            pltpu.sync_copy(labels_hbm.at[:, pl.ds(i0, BI)], slab_v)

            def pair(p, carry):
                for b in range(2):
                    q = p * 2 + b
                    j0 = q * JB

                    @pl.when(p > 0)
                    def _wait_out():
                        pltpu.make_async_copy(
                            quad_b[b],
                            out_hbm.at[pl.ds(0, JB), :, pl.ds(i0, BI)],
                            semo_b[b],
                        ).wait()

                    def jbody(jj, cc):
                        jrow(quad_b[b], jj, j0 + jj)
                        return cc

                    lax.fori_loop(0, JB, jbody, 0)
                    pltpu.async_copy(
                        quad_b[b],
                        out_hbm.at[pl.ds(j0, JB), :, pl.ds(i0, BI)],
                        semo_b[b],
                    )
                return carry

            lax.fori_loop(0, NQ // 2, pair, 0)
            for b in range(2):
                pltpu.make_async_copy(
                    quad_b[b],
                    out_hbm.at[pl.ds(0, JB), :, pl.ds(i0, BI)],
                    semo_b[b],
                ).wait()

    return k(labels_t, tabx)


def kernel(speaker_labels, table):
    t = table.at[0].set(0.0)
    tabx = jnp.zeros((D, L), jnp.float32).at[:, :3].set(t.T).reshape(-1)
    labels_t = speaker_labels.astype(jnp.int32).T
    out = _sc_lookup(labels_t, tabx)
    return out.transpose(2, 0, 1)
